# Initial kernel scaffold; baseline (speedup 1.0000x reference)
import functools

import jax
import jax.numpy as jnp
from jax import lax
from jax.experimental import pallas as pl
from jax.experimental.pallas import tpu as pltpu
from jax.experimental.pallas import tpu_sc as plsc

NNODE = 100000
NEDGE = 6400000
LANES = 128
ROWS = 784
NPAD = ROWS * LANES
NC = 2
NS = 16
NW = NC * NS
EPW = NEDGE // NW
BLK = 2000
NBLK = EPW // BLK
NSLICE = NPAD // NS

_mesh = plsc.VectorSubcoreMesh(core_axis_name="c", subcore_axis_name="s")


def _sc_gather_scatter(edge_hbm, y_hbm, zero_hbm, out_hbm,
                       idx_s, idx_d, vals, ysp, accsp, sem):
    core = lax.axis_index("c")
    sub = lax.axis_index("s")
    wid = core * NS + sub
    sl = pl.ds(sub * NSLICE, NSLICE)
    pltpu.sync_copy(y_hbm.at[sl], ysp.at[sl])
    pltpu.sync_copy(zero_hbm.at[sl], accsp.at[sl])
    plsc.subcore_barrier()

    def step(k, carry):
        base = wid * EPW + k * BLK
        pltpu.sync_copy(edge_hbm.at[0, pl.ds(base, BLK)], idx_s)
        pltpu.sync_copy(edge_hbm.at[1, pl.ds(base, BLK)], idx_d)
        pltpu.async_copy(ysp.at[idx_s], vals, sem).wait()
        pltpu.sync_copy(vals, accsp.at[idx_d], add=True)
        return carry

    lax.fori_loop(0, NBLK, step, 0)
    plsc.subcore_barrier()
    pltpu.sync_copy(accsp.at[sl], out_hbm.at[core, sl])


def _sc_degree(edge_hbm, one_hbm, zero_hbm, out_hbm, ones_v, idx_d, accsp, sem):
    core = lax.axis_index("c")
    sub = lax.axis_index("s")
    wid = core * NS + sub
    sl = pl.ds(sub * NSLICE, NSLICE)
    pltpu.sync_copy(zero_hbm.at[sl], accsp.at[sl])
    pltpu.sync_copy(one_hbm.at[:], ones_v)
    plsc.subcore_barrier()

    def step(k, carry):
        base = wid * EPW + k * BLK
        pltpu.sync_copy(edge_hbm.at[1, pl.ds(base, BLK)], idx_d)
        pltpu.sync_copy(ones_v, accsp.at[idx_d], add=True)
        return carry

    lax.fori_loop(0, NBLK, step, 0)
    plsc.subcore_barrier()
    pltpu.sync_copy(accsp.at[sl], out_hbm.at[core, sl])


_deg_call = functools.partial(
    pl.kernel,
    out_type=jax.ShapeDtypeStruct((NC, NPAD), jnp.float32),
    mesh=_mesh,
    scratch_types=[
        pltpu.VMEM((BLK,), jnp.float32),
        pltpu.VMEM((BLK,), jnp.int32),
        pltpu.VMEM_SHARED((NPAD,), jnp.float32),
        pltpu.SemaphoreType.DMA,
    ],
)(_sc_degree)

_pass_call = functools.partial(
    pl.kernel,
    out_type=jax.ShapeDtypeStruct((NC, NPAD), jnp.float32),
    mesh=_mesh,
    scratch_types=[
        pltpu.VMEM((BLK,), jnp.int32),
        pltpu.VMEM((BLK,), jnp.int32),
        pltpu.VMEM((BLK,), jnp.float32),
        pltpu.VMEM_SHARED((NPAD,), jnp.float32),
        pltpu.VMEM_SHARED((NPAD,), jnp.float32),
        pltpu.SemaphoreType.DMA,
    ],
)(_sc_gather_scatter)


def _tc_prep(degp_ref, x_ref, dinv_ref, y1_ref):
    deg = degp_ref[0] + degp_ref[1] + 1.0
    dinv = lax.rsqrt(deg)
    dinv_ref[...] = dinv
    y1_ref[...] = dinv * x_ref[...]


def _tc_mid(sp_ref, y_ref, dinv_ref, w_ref, b_ref, y2_ref):
    s = sp_ref[0] + sp_ref[1]
    conv = w_ref[0, 0] * dinv_ref[...] * (s + y_ref[...]) + b_ref[0, 0]
    y2_ref[...] = dinv_ref[...] * jnp.maximum(conv, 0.0)


def _tc_final(sp_ref, y_ref, dinv_ref, w_ref, b_ref, wl_ref, bl_ref, out_ref):
    s = sp_ref[0] + sp_ref[1]
    h2 = w_ref[0, 0] * dinv_ref[...] * (s + y_ref[...]) + b_ref[0, 0]
    out_ref[0, 0] = jnp.sum(h2 * wl_ref[...]) + bl_ref[0, 0]


def kernel(x, edge_index, W1, b1, W2, b2, Wl, bl):
    f32 = jnp.float32
    xp = jnp.pad(x[:, 0].astype(f32), (0, NPAD - NNODE))
    wlp = jnp.pad(Wl[0].astype(f32), (0, NPAD - NNODE))
    zero = jnp.zeros((NPAD,), f32)
    one_blk = jnp.ones((BLK,), f32)
    edge_index = edge_index.astype(jnp.int32)

    deg_p = _deg_call(edge_index, one_blk, zero)

    dinv, y1 = pl.pallas_call(
        _tc_prep,
        out_shape=(jax.ShapeDtypeStruct((ROWS, LANES), f32),
                   jax.ShapeDtypeStruct((ROWS, LANES), f32)),
    )(deg_p.reshape(NC, ROWS, LANES), xp.reshape(ROWS, LANES))

    s1_p = _pass_call(edge_index, y1.reshape(NPAD), zero)

    y2 = pl.pallas_call(
        _tc_mid,
        out_shape=jax.ShapeDtypeStruct((ROWS, LANES), f32),
    )(s1_p.reshape(NC, ROWS, LANES), y1, dinv,
      W1.astype(f32), b1.reshape(1, 1).astype(f32))

    s2_p = _pass_call(edge_index, y2.reshape(NPAD), zero)

    out = pl.pallas_call(
        _tc_final,
        out_shape=jax.ShapeDtypeStruct((1, 1), f32),
    )(s2_p.reshape(NC, ROWS, LANES), y2, dinv,
      W2.astype(f32), b2.reshape(1, 1).astype(f32),
      wlp.reshape(ROWS, LANES), bl.reshape(1, 1).astype(f32))

    return out


# speedup vs baseline: 275.8431x; 275.8431x over previous
import functools

import jax
import jax.numpy as jnp
from jax import lax
from jax.experimental import pallas as pl
from jax.experimental.pallas import tpu as pltpu
from jax.experimental.pallas import tpu_sc as plsc

NNODE = 100000
NEDGE = 6400000
LANES = 128
ROWS = 784
NPAD = ROWS * LANES
NC = 2
NS = 16
NW = NC * NS
EPW = NEDGE // NW
BLK = 2000
NBLK = EPW // BLK
NSLICE = NPAD // NS

_mesh = plsc.VectorSubcoreMesh(core_axis_name="c", subcore_axis_name="s")


def _sc_gather_scatter(edge_hbm, y_hbm, zero_hbm, out_hbm,
                       idx_s, idx_d, vals, ysp, accsp, sem):
    core = lax.axis_index("c")
    sub = lax.axis_index("s")
    wid = core * NS + sub
    sl = pl.ds(sub * NSLICE, NSLICE)
    pltpu.sync_copy(y_hbm.at[sl], ysp.at[sl])
    pltpu.sync_copy(zero_hbm.at[sl], accsp.at[sl])
    plsc.subcore_barrier()

    def step(k, carry):
        base = wid * EPW + k * BLK
        pltpu.sync_copy(edge_hbm.at[0, pl.ds(base, BLK)], idx_s)
        pltpu.sync_copy(edge_hbm.at[1, pl.ds(base, BLK)], idx_d)
        pltpu.async_copy(ysp.at[idx_s], vals, sem).wait()
        pltpu.sync_copy(vals, accsp.at[idx_d], add=True)
        return carry

    lax.fori_loop(0, NBLK, step, 0)
    plsc.subcore_barrier()
    pltpu.sync_copy(accsp.at[sl], out_hbm.at[core, sl])


def _sc_degree(edge_hbm, one_hbm, zero_hbm, out_hbm, ones_v, idx_d, accsp, sem):
    core = lax.axis_index("c")
    sub = lax.axis_index("s")
    wid = core * NS + sub
    sl = pl.ds(sub * NSLICE, NSLICE)
    pltpu.sync_copy(zero_hbm.at[sl], accsp.at[sl])
    pltpu.sync_copy(one_hbm.at[:], ones_v)
    plsc.subcore_barrier()

    def step(k, carry):
        base = wid * EPW + k * BLK
        pltpu.sync_copy(edge_hbm.at[1, pl.ds(base, BLK)], idx_d)
        pltpu.sync_copy(ones_v, accsp.at[idx_d], add=True)
        return carry

    lax.fori_loop(0, NBLK, step, 0)
    plsc.subcore_barrier()
    pltpu.sync_copy(accsp.at[sl], out_hbm.at[core, sl])


_sc_params = pltpu.CompilerParams(use_tc_tiling_on_sc=False)

_deg_call = functools.partial(
    pl.kernel,
    out_type=jax.ShapeDtypeStruct((NC, NPAD), jnp.float32),
    mesh=_mesh,
    compiler_params=_sc_params,
    scratch_types=[
        pltpu.VMEM((BLK,), jnp.float32),
        pltpu.VMEM((BLK,), jnp.int32),
        pltpu.VMEM_SHARED((NPAD,), jnp.float32),
        pltpu.SemaphoreType.DMA,
    ],
)(_sc_degree)

_pass_call = functools.partial(
    pl.kernel,
    out_type=jax.ShapeDtypeStruct((NC, NPAD), jnp.float32),
    mesh=_mesh,
    compiler_params=_sc_params,
    scratch_types=[
        pltpu.VMEM((BLK,), jnp.int32),
        pltpu.VMEM((BLK,), jnp.int32),
        pltpu.VMEM((BLK,), jnp.float32),
        pltpu.VMEM_SHARED((NPAD,), jnp.float32),
        pltpu.VMEM_SHARED((NPAD,), jnp.float32),
        pltpu.SemaphoreType.DMA,
    ],
)(_sc_gather_scatter)


def _tc_prep(degp_ref, x_ref, dinv_ref, y1_ref):
    deg = degp_ref[0] + degp_ref[1] + 1.0
    dinv = lax.rsqrt(deg)
    dinv_ref[...] = dinv
    y1_ref[...] = dinv * x_ref[...]


def _tc_mid(sp_ref, y_ref, dinv_ref, w_ref, b_ref, y2_ref):
    s = sp_ref[0] + sp_ref[1]
    conv = w_ref[0, 0] * dinv_ref[...] * (s + y_ref[...]) + b_ref[0, 0]
    y2_ref[...] = dinv_ref[...] * jnp.maximum(conv, 0.0)


def _tc_final(sp_ref, y_ref, dinv_ref, w_ref, b_ref, wl_ref, bl_ref, out_ref):
    s = sp_ref[0] + sp_ref[1]
    h2 = w_ref[0, 0] * dinv_ref[...] * (s + y_ref[...]) + b_ref[0, 0]
    out_ref[...] = jnp.sum(h2 * wl_ref[...], keepdims=True) + bl_ref[...]


def kernel(x, edge_index, W1, b1, W2, b2, Wl, bl):
    f32 = jnp.float32
    xp = jnp.pad(x[:, 0].astype(f32), (0, NPAD - NNODE))
    wlp = jnp.pad(Wl[0].astype(f32), (0, NPAD - NNODE))
    zero = jnp.zeros((NPAD,), f32)
    one_blk = jnp.ones((BLK,), f32)
    edge_index = edge_index.astype(jnp.int32)

    deg_p = _deg_call(edge_index, one_blk, zero)

    dinv, y1 = pl.pallas_call(
        _tc_prep,
        out_shape=(jax.ShapeDtypeStruct((ROWS, LANES), f32),
                   jax.ShapeDtypeStruct((ROWS, LANES), f32)),
    )(deg_p.reshape(NC, ROWS, LANES), xp.reshape(ROWS, LANES))

    s1_p = _pass_call(edge_index, y1.reshape(NPAD), zero)

    y2 = pl.pallas_call(
        _tc_mid,
        out_shape=jax.ShapeDtypeStruct((ROWS, LANES), f32),
    )(s1_p.reshape(NC, ROWS, LANES), y1, dinv,
      W1.astype(f32), b1.reshape(1, 1).astype(f32))

    s2_p = _pass_call(edge_index, y2.reshape(NPAD), zero)

    out = pl.pallas_call(
        _tc_final,
        out_shape=jax.ShapeDtypeStruct((1, 1), f32),
    )(s2_p.reshape(NC, ROWS, LANES), y2, dinv,
      W2.astype(f32), b2.reshape(1, 1).astype(f32),
      wlp.reshape(ROWS, LANES), bl.reshape(1, 1).astype(f32))

    return out


# speedup vs baseline: 545.1177x; 1.9762x over previous
import functools

import jax
import jax.numpy as jnp
from jax import lax
from jax.experimental import pallas as pl
from jax.experimental.pallas import tpu as pltpu
from jax.experimental.pallas import tpu_sc as plsc

NNODE = 100000
NEDGE = 6400000
LANES = 128
ROWS = 784
NPAD = ROWS * LANES
NC = 2
NS = 16
NW = NC * NS
EPW = NEDGE // NW
BLK = 2000
NBLK = EPW // BLK
NSLICE = NPAD // NS

_mesh = plsc.VectorSubcoreMesh(core_axis_name="c", subcore_axis_name="s")


GROUPS = BLK // 16


def _gather_block(yv, idx_ref, vals_ref):
    def body(i, carry):
        sl = pl.ds(i * 16, 16)
        vals_ref[sl] = plsc.load_gather(yv, [idx_ref[sl]])
        return carry

    lax.fori_loop(0, GROUPS, body, 0, unroll=5)


def _sc_gather_scatter(edge_hbm, y_hbm, zero_hbm, out_hbm,
                       idx_s0, idx_s1, idx_d0, idx_d1, idx_d2, idx_d3,
                       vals0, vals1, yv, accsp,
                       sem_i0, sem_i1, sem_s0, sem_s1):
    core = lax.axis_index("c")
    sub = lax.axis_index("s")
    wid = core * NS + sub
    ebase = wid * EPW
    sl = pl.ds(sub * NSLICE, NSLICE)
    idx_s = (idx_s0, idx_s1)
    idx_d = (idx_d0, idx_d1, idx_d2, idx_d3)
    vals = (vals0, vals1)
    sem_i = (sem_i0, sem_i1)
    sem_s = (sem_s0, sem_s1)

    pltpu.sync_copy(y_hbm, yv)
    pltpu.sync_copy(zero_hbm.at[sl], accsp.at[sl])
    plsc.subcore_barrier()

    def start_idx(k, b2, b4):
        base = ebase + k * BLK
        pltpu.async_copy(edge_hbm.at[0, pl.ds(base, BLK)], idx_s[b2], sem_i[b2])
        pltpu.async_copy(edge_hbm.at[1, pl.ds(base, BLK)], idx_d[b4], sem_i[b2])

    def wait_idx(b2, b4):
        pltpu.make_async_copy(edge_hbm.at[0, pl.ds(0, BLK)], idx_s[b2],
                              sem_i[b2]).wait()
        pltpu.make_async_copy(edge_hbm.at[1, pl.ds(0, BLK)], idx_d[b4],
                              sem_i[b2]).wait()

    def wait_scat(b2, b4):
        pltpu.make_async_copy(vals[b2], accsp.at[idx_d[b4]], sem_s[b2]).wait()

    start_idx(0, 0, 0)

    def outer(j, carry):
        for b in range(4):
            k = 4 * j + b
            b2 = b % 2
            if b >= 2:
                wait_scat(b2, b - 2)
            else:
                @pl.when(j >= 1)
                def _():
                    wait_scat(b2, b + 2)
            if b < 3:
                start_idx(k + 1, 1 - b2, b + 1)
            else:
                @pl.when(j < (NBLK // 4) - 1)
                def _():
                    start_idx(k + 1, 1 - b2, 0)
            wait_idx(b2, b)
            _gather_block(yv, idx_s[b2], vals[b2])
            pltpu.async_copy(vals[b2], accsp.at[idx_d[b]], sem_s[b2], add=True)
        return carry

    lax.fori_loop(0, NBLK // 4, outer, 0)
    wait_scat(0, 2)
    wait_scat(1, 3)
    plsc.subcore_barrier()
    pltpu.sync_copy(accsp.at[sl], out_hbm.at[core, sl])


def _sc_degree(edge_hbm, one_hbm, zero_hbm, out_hbm,
               ones_v, idx_d0, idx_d1, idx_d2, idx_d3, accsp,
               sem_i0, sem_i1, sem_s0, sem_s1):
    core = lax.axis_index("c")
    sub = lax.axis_index("s")
    wid = core * NS + sub
    ebase = wid * EPW
    sl = pl.ds(sub * NSLICE, NSLICE)
    idx_d = (idx_d0, idx_d1, idx_d2, idx_d3)
    sem_i = (sem_i0, sem_i1)
    sem_s = (sem_s0, sem_s1)

    pltpu.sync_copy(zero_hbm.at[sl], accsp.at[sl])
    pltpu.sync_copy(one_hbm, ones_v)
    plsc.subcore_barrier()

    def start_idx(k, b2, b4):
        base = ebase + k * BLK
        pltpu.async_copy(edge_hbm.at[1, pl.ds(base, BLK)], idx_d[b4], sem_i[b2])

    def wait_idx(b2, b4):
        pltpu.make_async_copy(edge_hbm.at[1, pl.ds(0, BLK)], idx_d[b4],
                              sem_i[b2]).wait()

    def wait_scat(b2, b4):
        pltpu.make_async_copy(ones_v, accsp.at[idx_d[b4]], sem_s[b2]).wait()

    start_idx(0, 0, 0)

    def outer(j, carry):
        for b in range(4):
            k = 4 * j + b
            b2 = b % 2
            if b >= 2:
                wait_scat(b2, b - 2)
            else:
                @pl.when(j >= 1)
                def _():
                    wait_scat(b2, b + 2)
            if b < 3:
                start_idx(k + 1, 1 - b2, b + 1)
            else:
                @pl.when(j < (NBLK // 4) - 1)
                def _():
                    start_idx(k + 1, 1 - b2, 0)
            wait_idx(b2, b)
            pltpu.async_copy(ones_v, accsp.at[idx_d[b]], sem_s[b2], add=True)
        return carry

    lax.fori_loop(0, NBLK // 4, outer, 0)
    wait_scat(0, 2)
    wait_scat(1, 3)
    plsc.subcore_barrier()
    pltpu.sync_copy(accsp.at[sl], out_hbm.at[core, sl])


_sc_params = pltpu.CompilerParams(use_tc_tiling_on_sc=False, needs_layout_passes=False)

_deg_call = functools.partial(
    pl.kernel,
    out_type=jax.ShapeDtypeStruct((NC, NPAD), jnp.float32),
    mesh=_mesh,
    compiler_params=_sc_params,
    scratch_types=[
        pltpu.VMEM((BLK,), jnp.float32),
        pltpu.VMEM((BLK,), jnp.int32),
        pltpu.VMEM((BLK,), jnp.int32),
        pltpu.VMEM((BLK,), jnp.int32),
        pltpu.VMEM((BLK,), jnp.int32),
        pltpu.VMEM_SHARED((NPAD,), jnp.float32),
        pltpu.SemaphoreType.DMA,
        pltpu.SemaphoreType.DMA,
        pltpu.SemaphoreType.DMA,
        pltpu.SemaphoreType.DMA,
    ],
)(_sc_degree)

_pass_call = functools.partial(
    pl.kernel,
    out_type=jax.ShapeDtypeStruct((NC, NPAD), jnp.float32),
    mesh=_mesh,
    compiler_params=_sc_params,
    scratch_types=[
        pltpu.VMEM((BLK,), jnp.int32),
        pltpu.VMEM((BLK,), jnp.int32),
        pltpu.VMEM((BLK,), jnp.int32),
        pltpu.VMEM((BLK,), jnp.int32),
        pltpu.VMEM((BLK,), jnp.int32),
        pltpu.VMEM((BLK,), jnp.int32),
        pltpu.VMEM((BLK,), jnp.float32),
        pltpu.VMEM((BLK,), jnp.float32),
        pltpu.VMEM((NPAD,), jnp.float32),
        pltpu.VMEM_SHARED((NPAD,), jnp.float32),
        pltpu.SemaphoreType.DMA,
        pltpu.SemaphoreType.DMA,
        pltpu.SemaphoreType.DMA,
        pltpu.SemaphoreType.DMA,
    ],
)(_sc_gather_scatter)


def _tc_prep(degp_ref, x_ref, dinv_ref, y1_ref):
    deg = degp_ref[0] + degp_ref[1] + 1.0
    dinv = lax.rsqrt(deg)
    dinv_ref[...] = dinv
    y1_ref[...] = dinv * x_ref[...]


def _tc_mid(sp_ref, y_ref, dinv_ref, w_ref, b_ref, y2_ref):
    s = sp_ref[0] + sp_ref[1]
    conv = w_ref[0, 0] * dinv_ref[...] * (s + y_ref[...]) + b_ref[0, 0]
    y2_ref[...] = dinv_ref[...] * jnp.maximum(conv, 0.0)


def _tc_final(sp_ref, y_ref, dinv_ref, w_ref, b_ref, wl_ref, bl_ref, out_ref):
    s = sp_ref[0] + sp_ref[1]
    h2 = w_ref[0, 0] * dinv_ref[...] * (s + y_ref[...]) + b_ref[0, 0]
    out_ref[...] = jnp.sum(h2 * wl_ref[...], keepdims=True) + bl_ref[...]


def kernel(x, edge_index, W1, b1, W2, b2, Wl, bl):
    f32 = jnp.float32
    xp = jnp.pad(x[:, 0].astype(f32), (0, NPAD - NNODE))
    wlp = jnp.pad(Wl[0].astype(f32), (0, NPAD - NNODE))
    zero = jnp.zeros((NPAD,), f32)
    one_blk = jnp.ones((BLK,), f32)
    edge_index = edge_index.astype(jnp.int32)

    deg_p = _deg_call(edge_index, one_blk, zero)

    dinv, y1 = pl.pallas_call(
        _tc_prep,
        out_shape=(jax.ShapeDtypeStruct((ROWS, LANES), f32),
                   jax.ShapeDtypeStruct((ROWS, LANES), f32)),
    )(deg_p.reshape(NC, ROWS, LANES), xp.reshape(ROWS, LANES))

    s1_p = _pass_call(edge_index, y1.reshape(NPAD), zero)

    y2 = pl.pallas_call(
        _tc_mid,
        out_shape=jax.ShapeDtypeStruct((ROWS, LANES), f32),
    )(s1_p.reshape(NC, ROWS, LANES), y1, dinv,
      W1.astype(f32), b1.reshape(1, 1).astype(f32))

    s2_p = _pass_call(edge_index, y2.reshape(NPAD), zero)

    out = pl.pallas_call(
        _tc_final,
        out_shape=jax.ShapeDtypeStruct((1, 1), f32),
    )(s2_p.reshape(NC, ROWS, LANES), y2, dinv,
      W2.astype(f32), b2.reshape(1, 1).astype(f32),
      wlp.reshape(ROWS, LANES), bl.reshape(1, 1).astype(f32))

    return out


# speedup vs baseline: 732.5643x; 1.3439x over previous
import functools

import jax
import jax.numpy as jnp
from jax import lax
from jax.experimental import pallas as pl
from jax.experimental.pallas import tpu as pltpu
from jax.experimental.pallas import tpu_sc as plsc

NNODE = 100000
NEDGE = 6400000
LANES = 128
ROWS = 784
NPAD = ROWS * LANES
NC = 2
NS = 16
NW = NC * NS
EPW = NEDGE // NW
BLK = 2000
NBLK = EPW // BLK
NSLICE = NPAD // NS

_mesh = plsc.VectorSubcoreMesh(core_axis_name="c", subcore_axis_name="s")


GROUPS = BLK // 16


def _gather_block(yv, idx_ref, vals_ref):
    @plsc.parallel_loop(0, BLK, step=16, unroll=8)
    def _(i):
        sl = pl.ds(i, 16)
        vals_ref[sl] = plsc.load_gather(yv, [idx_ref[sl]])


def _sc_gather_scatter(edge_hbm, y_hbm, zero_hbm, out_hbm,
                       idx_s0, idx_s1, idx_d0, idx_d1, idx_d2, idx_d3,
                       vals0, vals1, yv, accsp,
                       sem_i0, sem_i1, sem_s0, sem_s1):
    core = lax.axis_index("c")
    sub = lax.axis_index("s")
    wid = core * NS + sub
    ebase = wid * EPW
    sl = pl.ds(sub * NSLICE, NSLICE)
    idx_s = (idx_s0, idx_s1)
    idx_d = (idx_d0, idx_d1, idx_d2, idx_d3)
    vals = (vals0, vals1)
    sem_i = (sem_i0, sem_i1)
    sem_s = (sem_s0, sem_s1)

    pltpu.sync_copy(y_hbm, yv)
    pltpu.sync_copy(zero_hbm.at[sl], accsp.at[sl])
    plsc.subcore_barrier()

    def start_idx(k, b2, b4):
        base = ebase + k * BLK
        pltpu.async_copy(edge_hbm.at[0, pl.ds(base, BLK)], idx_s[b2], sem_i[b2])
        pltpu.async_copy(edge_hbm.at[1, pl.ds(base, BLK)], idx_d[b4], sem_i[b2])

    def wait_idx(b2, b4):
        pltpu.make_async_copy(edge_hbm.at[0, pl.ds(0, BLK)], idx_s[b2],
                              sem_i[b2]).wait()
        pltpu.make_async_copy(edge_hbm.at[1, pl.ds(0, BLK)], idx_d[b4],
                              sem_i[b2]).wait()

    def wait_scat(b2, b4):
        pltpu.make_async_copy(vals[b2], accsp.at[idx_d[b4]], sem_s[b2]).wait()

    start_idx(0, 0, 0)

    def outer(j, carry):
        for b in range(4):
            k = 4 * j + b
            b2 = b % 2
            if b >= 2:
                wait_scat(b2, b - 2)
            else:
                @pl.when(j >= 1)
                def _():
                    wait_scat(b2, b + 2)
            if b < 3:
                start_idx(k + 1, 1 - b2, b + 1)
            else:
                @pl.when(j < (NBLK // 4) - 1)
                def _():
                    start_idx(k + 1, 1 - b2, 0)
            wait_idx(b2, b)
            _gather_block(yv, idx_s[b2], vals[b2])
            pltpu.async_copy(vals[b2], accsp.at[idx_d[b]], sem_s[b2], add=True)
        return carry

    lax.fori_loop(0, NBLK // 4, outer, 0)
    wait_scat(0, 2)
    wait_scat(1, 3)
    plsc.subcore_barrier()
    pltpu.sync_copy(accsp.at[sl], out_hbm.at[core, sl])


def _sc_degree(edge_hbm, one_hbm, zero_hbm, out_hbm,
               ones_v, idx_d0, idx_d1, idx_d2, idx_d3, accsp,
               sem_i0, sem_i1, sem_s0, sem_s1):
    core = lax.axis_index("c")
    sub = lax.axis_index("s")
    wid = core * NS + sub
    ebase = wid * EPW
    sl = pl.ds(sub * NSLICE, NSLICE)
    idx_d = (idx_d0, idx_d1, idx_d2, idx_d3)
    sem_i = (sem_i0, sem_i1)
    sem_s = (sem_s0, sem_s1)

    pltpu.sync_copy(zero_hbm.at[sl], accsp.at[sl])
    pltpu.sync_copy(one_hbm, ones_v)
    plsc.subcore_barrier()

    def start_idx(k, b2, b4):
        base = ebase + k * BLK
        pltpu.async_copy(edge_hbm.at[1, pl.ds(base, BLK)], idx_d[b4], sem_i[b2])

    def wait_idx(b2, b4):
        pltpu.make_async_copy(edge_hbm.at[1, pl.ds(0, BLK)], idx_d[b4],
                              sem_i[b2]).wait()

    def wait_scat(b2, b4):
        pltpu.make_async_copy(ones_v, accsp.at[idx_d[b4]], sem_s[b2]).wait()

    start_idx(0, 0, 0)

    def outer(j, carry):
        for b in range(4):
            k = 4 * j + b
            b2 = b % 2
            if b >= 2:
                wait_scat(b2, b - 2)
            else:
                @pl.when(j >= 1)
                def _():
                    wait_scat(b2, b + 2)
            if b < 3:
                start_idx(k + 1, 1 - b2, b + 1)
            else:
                @pl.when(j < (NBLK // 4) - 1)
                def _():
                    start_idx(k + 1, 1 - b2, 0)
            wait_idx(b2, b)
            pltpu.async_copy(ones_v, accsp.at[idx_d[b]], sem_s[b2], add=True)
        return carry

    lax.fori_loop(0, NBLK // 4, outer, 0)
    wait_scat(0, 2)
    wait_scat(1, 3)
    plsc.subcore_barrier()
    pltpu.sync_copy(accsp.at[sl], out_hbm.at[core, sl])


_sc_params = pltpu.CompilerParams(use_tc_tiling_on_sc=False, needs_layout_passes=False)

_deg_call = functools.partial(
    pl.kernel,
    out_type=jax.ShapeDtypeStruct((NC, NPAD), jnp.float32),
    mesh=_mesh,
    compiler_params=_sc_params,
    scratch_types=[
        pltpu.VMEM((BLK,), jnp.float32),
        pltpu.VMEM((BLK,), jnp.int32),
        pltpu.VMEM((BLK,), jnp.int32),
        pltpu.VMEM((BLK,), jnp.int32),
        pltpu.VMEM((BLK,), jnp.int32),
        pltpu.VMEM_SHARED((NPAD,), jnp.float32),
        pltpu.SemaphoreType.DMA,
        pltpu.SemaphoreType.DMA,
        pltpu.SemaphoreType.DMA,
        pltpu.SemaphoreType.DMA,
    ],
)(_sc_degree)

_pass_call = functools.partial(
    pl.kernel,
    out_type=jax.ShapeDtypeStruct((NC, NPAD), jnp.float32),
    mesh=_mesh,
    compiler_params=_sc_params,
    scratch_types=[
        pltpu.VMEM((BLK,), jnp.int32),
        pltpu.VMEM((BLK,), jnp.int32),
        pltpu.VMEM((BLK,), jnp.int32),
        pltpu.VMEM((BLK,), jnp.int32),
        pltpu.VMEM((BLK,), jnp.int32),
        pltpu.VMEM((BLK,), jnp.int32),
        pltpu.VMEM((BLK,), jnp.float32),
        pltpu.VMEM((BLK,), jnp.float32),
        pltpu.VMEM((NPAD,), jnp.float32),
        pltpu.VMEM_SHARED((NPAD,), jnp.float32),
        pltpu.SemaphoreType.DMA,
        pltpu.SemaphoreType.DMA,
        pltpu.SemaphoreType.DMA,
        pltpu.SemaphoreType.DMA,
    ],
)(_sc_gather_scatter)


def _tc_prep(degp_ref, x_ref, dinv_ref, y1_ref):
    deg = degp_ref[0] + degp_ref[1] + 1.0
    dinv = lax.rsqrt(deg)
    dinv_ref[...] = dinv
    y1_ref[...] = dinv * x_ref[...]


def _tc_mid(sp_ref, y_ref, dinv_ref, w_ref, b_ref, y2_ref):
    s = sp_ref[0] + sp_ref[1]
    conv = w_ref[0, 0] * dinv_ref[...] * (s + y_ref[...]) + b_ref[0, 0]
    y2_ref[...] = dinv_ref[...] * jnp.maximum(conv, 0.0)


def _tc_final(sp_ref, y_ref, dinv_ref, w_ref, b_ref, wl_ref, bl_ref, out_ref):
    s = sp_ref[0] + sp_ref[1]
    h2 = w_ref[0, 0] * dinv_ref[...] * (s + y_ref[...]) + b_ref[0, 0]
    out_ref[...] = jnp.sum(h2 * wl_ref[...], keepdims=True) + bl_ref[...]


def kernel(x, edge_index, W1, b1, W2, b2, Wl, bl):
    f32 = jnp.float32
    xp = jnp.pad(x[:, 0].astype(f32), (0, NPAD - NNODE))
    wlp = jnp.pad(Wl[0].astype(f32), (0, NPAD - NNODE))
    zero = jnp.zeros((NPAD,), f32)
    one_blk = jnp.ones((BLK,), f32)
    edge_index = edge_index.astype(jnp.int32)

    deg_p = _deg_call(edge_index, one_blk, zero)

    dinv, y1 = pl.pallas_call(
        _tc_prep,
        out_shape=(jax.ShapeDtypeStruct((ROWS, LANES), f32),
                   jax.ShapeDtypeStruct((ROWS, LANES), f32)),
    )(deg_p.reshape(NC, ROWS, LANES), xp.reshape(ROWS, LANES))

    s1_p = _pass_call(edge_index, y1.reshape(NPAD), zero)

    y2 = pl.pallas_call(
        _tc_mid,
        out_shape=jax.ShapeDtypeStruct((ROWS, LANES), f32),
    )(s1_p.reshape(NC, ROWS, LANES), y1, dinv,
      W1.astype(f32), b1.reshape(1, 1).astype(f32))

    s2_p = _pass_call(edge_index, y2.reshape(NPAD), zero)

    out = pl.pallas_call(
        _tc_final,
        out_shape=jax.ShapeDtypeStruct((1, 1), f32),
    )(s2_p.reshape(NC, ROWS, LANES), y2, dinv,
      W2.astype(f32), b2.reshape(1, 1).astype(f32),
      wlp.reshape(ROWS, LANES), bl.reshape(1, 1).astype(f32))

    return out


# speedup vs baseline: 765.7065x; 1.0452x over previous
import functools

import jax
import jax.numpy as jnp
from jax import lax
from jax.experimental import pallas as pl
from jax.experimental.pallas import tpu as pltpu
from jax.experimental.pallas import tpu_sc as plsc

NNODE = 100000
NEDGE = 6400000
LANES = 128
ROWS = 784
NPAD = ROWS * LANES
NC = 2
NS = 16
NW = NC * NS
EPW = NEDGE // NW
BLK = 2000
NBLK = EPW // BLK
NSLICE = NPAD // NS

_mesh = plsc.VectorSubcoreMesh(core_axis_name="c", subcore_axis_name="s")


GROUPS = BLK // 16


def _gather_block(yv, idx_ref, vals_ref):
    @plsc.parallel_loop(0, BLK, step=16, unroll=16)
    def _(i):
        sl = pl.ds(i, 16)
        vals_ref[sl] = plsc.load_gather(yv, [idx_ref[sl]])


def _sc_gather_scatter(edge_hbm, y_hbm, zero_hbm, out_hbm,
                       idx_s0, idx_s1, idx_d0, idx_d1, idx_d2, idx_d3,
                       vals0, vals1, vals2, vals3, yv, accsp,
                       sem_i0, sem_i1, sem_s0, sem_s1, sem_s2, sem_s3):
    core = lax.axis_index("c")
    sub = lax.axis_index("s")
    wid = core * NS + sub
    ebase = wid * EPW
    sl = pl.ds(sub * NSLICE, NSLICE)
    idx_s = (idx_s0, idx_s1)
    idx_d = (idx_d0, idx_d1, idx_d2, idx_d3)
    vals = (vals0, vals1, vals2, vals3)
    sem_i = (sem_i0, sem_i1)
    sem_s = (sem_s0, sem_s1, sem_s2, sem_s3)

    pltpu.sync_copy(y_hbm, yv)
    pltpu.sync_copy(zero_hbm.at[sl], accsp.at[sl])
    plsc.subcore_barrier()

    def start_idx(k, b2, b4):
        base = ebase + k * BLK
        pltpu.async_copy(edge_hbm.at[0, pl.ds(base, BLK)], idx_s[b2], sem_i[b2])
        pltpu.async_copy(edge_hbm.at[1, pl.ds(base, BLK)], idx_d[b4], sem_i[b2])

    def wait_idx(b2, b4):
        pltpu.make_async_copy(edge_hbm.at[0, pl.ds(0, BLK)], idx_s[b2],
                              sem_i[b2]).wait()
        pltpu.make_async_copy(edge_hbm.at[1, pl.ds(0, BLK)], idx_d[b4],
                              sem_i[b2]).wait()

    def wait_scat(b4):
        pltpu.make_async_copy(vals[b4], accsp.at[idx_d[b4]], sem_s[b4]).wait()

    start_idx(0, 0, 0)

    def outer(j, carry):
        for b in range(4):
            k = 4 * j + b
            b2 = b % 2
            if b == 3:
                wait_scat(0)
            else:
                @pl.when(j >= 1)
                def _():
                    wait_scat(b + 1)
            if b < 3:
                start_idx(k + 1, 1 - b2, b + 1)
            else:
                @pl.when(j < (NBLK // 4) - 1)
                def _():
                    start_idx(k + 1, 1 - b2, 0)
            wait_idx(b2, b)
            _gather_block(yv, idx_s[b2], vals[b])
            pltpu.async_copy(vals[b], accsp.at[idx_d[b]], sem_s[b], add=True)
        return carry

    lax.fori_loop(0, NBLK // 4, outer, 0)
    wait_scat(1)
    wait_scat(2)
    wait_scat(3)
    plsc.subcore_barrier()
    pltpu.sync_copy(accsp.at[sl], out_hbm.at[core, sl])


def _sc_degree(edge_hbm, one_hbm, zero_hbm, out_hbm,
               ones_v, idx_d0, idx_d1, idx_d2, idx_d3, accsp,
               sem_i0, sem_i1, sem_s0, sem_s1, sem_s2, sem_s3):
    core = lax.axis_index("c")
    sub = lax.axis_index("s")
    wid = core * NS + sub
    ebase = wid * EPW
    sl = pl.ds(sub * NSLICE, NSLICE)
    idx_d = (idx_d0, idx_d1, idx_d2, idx_d3)
    sem_i = (sem_i0, sem_i1)
    sem_s = (sem_s0, sem_s1, sem_s2, sem_s3)

    pltpu.sync_copy(zero_hbm.at[sl], accsp.at[sl])
    pltpu.sync_copy(one_hbm, ones_v)
    plsc.subcore_barrier()

    def start_idx(k, b2, b4):
        base = ebase + k * BLK
        pltpu.async_copy(edge_hbm.at[1, pl.ds(base, BLK)], idx_d[b4], sem_i[b2])

    def wait_idx(b2, b4):
        pltpu.make_async_copy(edge_hbm.at[1, pl.ds(0, BLK)], idx_d[b4],
                              sem_i[b2]).wait()

    def wait_scat(b4):
        pltpu.make_async_copy(ones_v, accsp.at[idx_d[b4]], sem_s[b4]).wait()

    start_idx(0, 0, 0)

    def outer(j, carry):
        for b in range(4):
            k = 4 * j + b
            b2 = b % 2
            if b == 3:
                wait_scat(0)
            else:
                @pl.when(j >= 1)
                def _():
                    wait_scat(b + 1)
            if b < 3:
                start_idx(k + 1, 1 - b2, b + 1)
            else:
                @pl.when(j < (NBLK // 4) - 1)
                def _():
                    start_idx(k + 1, 1 - b2, 0)
            wait_idx(b2, b)
            pltpu.async_copy(ones_v, accsp.at[idx_d[b]], sem_s[b], add=True)
        return carry

    lax.fori_loop(0, NBLK // 4, outer, 0)
    wait_scat(1)
    wait_scat(2)
    wait_scat(3)
    plsc.subcore_barrier()
    pltpu.sync_copy(accsp.at[sl], out_hbm.at[core, sl])


_sc_params = pltpu.CompilerParams(use_tc_tiling_on_sc=False, needs_layout_passes=False)

_deg_call = functools.partial(
    pl.kernel,
    out_type=jax.ShapeDtypeStruct((NC, NPAD), jnp.float32),
    mesh=_mesh,
    compiler_params=_sc_params,
    scratch_types=[
        pltpu.VMEM((BLK,), jnp.float32),
        pltpu.VMEM((BLK,), jnp.int32),
        pltpu.VMEM((BLK,), jnp.int32),
        pltpu.VMEM((BLK,), jnp.int32),
        pltpu.VMEM((BLK,), jnp.int32),
        pltpu.VMEM_SHARED((NPAD,), jnp.float32),
        pltpu.SemaphoreType.DMA,
        pltpu.SemaphoreType.DMA,
        pltpu.SemaphoreType.DMA,
        pltpu.SemaphoreType.DMA,
        pltpu.SemaphoreType.DMA,
        pltpu.SemaphoreType.DMA,
    ],
)(_sc_degree)

_pass_call = functools.partial(
    pl.kernel,
    out_type=jax.ShapeDtypeStruct((NC, NPAD), jnp.float32),
    mesh=_mesh,
    compiler_params=_sc_params,
    scratch_types=[
        pltpu.VMEM((BLK,), jnp.int32),
        pltpu.VMEM((BLK,), jnp.int32),
        pltpu.VMEM((BLK,), jnp.int32),
        pltpu.VMEM((BLK,), jnp.int32),
        pltpu.VMEM((BLK,), jnp.int32),
        pltpu.VMEM((BLK,), jnp.int32),
        pltpu.VMEM((BLK,), jnp.float32),
        pltpu.VMEM((BLK,), jnp.float32),
        pltpu.VMEM((BLK,), jnp.float32),
        pltpu.VMEM((BLK,), jnp.float32),
        pltpu.VMEM((NPAD,), jnp.float32),
        pltpu.VMEM_SHARED((NPAD,), jnp.float32),
        pltpu.SemaphoreType.DMA,
        pltpu.SemaphoreType.DMA,
        pltpu.SemaphoreType.DMA,
        pltpu.SemaphoreType.DMA,
        pltpu.SemaphoreType.DMA,
        pltpu.SemaphoreType.DMA,
    ],
)(_sc_gather_scatter)


def _tc_prep(degp_ref, x_ref, dinv_ref, y1_ref):
    deg = degp_ref[0] + degp_ref[1] + 1.0
    dinv = lax.rsqrt(deg)
    dinv_ref[...] = dinv
    y1_ref[...] = dinv * x_ref[...]


def _tc_mid(sp_ref, y_ref, dinv_ref, w_ref, b_ref, y2_ref):
    s = sp_ref[0] + sp_ref[1]
    conv = w_ref[0, 0] * dinv_ref[...] * (s + y_ref[...]) + b_ref[0, 0]
    y2_ref[...] = dinv_ref[...] * jnp.maximum(conv, 0.0)


def _tc_final(sp_ref, y_ref, dinv_ref, w_ref, b_ref, wl_ref, bl_ref, out_ref):
    s = sp_ref[0] + sp_ref[1]
    h2 = w_ref[0, 0] * dinv_ref[...] * (s + y_ref[...]) + b_ref[0, 0]
    out_ref[...] = jnp.sum(h2 * wl_ref[...], keepdims=True) + bl_ref[...]


def kernel(x, edge_index, W1, b1, W2, b2, Wl, bl):
    f32 = jnp.float32
    xp = jnp.pad(x[:, 0].astype(f32), (0, NPAD - NNODE))
    wlp = jnp.pad(Wl[0].astype(f32), (0, NPAD - NNODE))
    zero = jnp.zeros((NPAD,), f32)
    one_blk = jnp.ones((BLK,), f32)
    edge_index = edge_index.astype(jnp.int32)

    deg_p = _deg_call(edge_index, one_blk, zero)

    dinv, y1 = pl.pallas_call(
        _tc_prep,
        out_shape=(jax.ShapeDtypeStruct((ROWS, LANES), f32),
                   jax.ShapeDtypeStruct((ROWS, LANES), f32)),
    )(deg_p.reshape(NC, ROWS, LANES), xp.reshape(ROWS, LANES))

    s1_p = _pass_call(edge_index, y1.reshape(NPAD), zero)

    y2 = pl.pallas_call(
        _tc_mid,
        out_shape=jax.ShapeDtypeStruct((ROWS, LANES), f32),
    )(s1_p.reshape(NC, ROWS, LANES), y1, dinv,
      W1.astype(f32), b1.reshape(1, 1).astype(f32))

    s2_p = _pass_call(edge_index, y2.reshape(NPAD), zero)

    out = pl.pallas_call(
        _tc_final,
        out_shape=jax.ShapeDtypeStruct((1, 1), f32),
    )(s2_p.reshape(NC, ROWS, LANES), y2, dinv,
      W2.astype(f32), b2.reshape(1, 1).astype(f32),
      wlp.reshape(ROWS, LANES), bl.reshape(1, 1).astype(f32))

    return out
